# baseline (speedup 1.0000x reference)
import functools

import jax
import jax.numpy as jnp
from jax import lax
from jax.experimental import pallas as pl
from jax.experimental.pallas import tpu as pltpu
from jax.experimental.pallas import tpu_sc as plsc

NP_ = 32
NK_ = 32
M_ = NP_ * NK_
EMBED_ = 256
BR_ = 2048


def _tc_body(q_ref, P_ref, pred_ref, D_ref):
    q = q_ref[...]
    Pm = P_ref[...]
    qb = (q * -2.0).astype(jnp.bfloat16)
    Pb = Pm.astype(jnp.bfloat16)
    S = jax.lax.dot_general(
        qb, Pb, (((1,), (1,)), ((), ())),
        preferred_element_type=jnp.float32)
    q2 = jnp.sum(q * q, axis=1, keepdims=True)
    p2 = jnp.sum(Pm * Pm, axis=1)[None, :]
    m = jnp.maximum((q2 + p2) + S, 1e-12)
    D = m * jax.lax.rsqrt(m)
    E = jnp.exp(-D)
    D_ref[...] = D
    col = jax.lax.broadcasted_iota(jnp.int32, (M_, NP_), 0)
    grp = jax.lax.broadcasted_iota(jnp.int32, (M_, NP_), 1)
    G = jnp.where(col // NK_ == grp, 1.0, 0.0).astype(jnp.float32)
    ps = jax.lax.dot_general(
        E, G, (((1,), (0,)), ((), ())),
        preferred_element_type=jnp.float32)
    pred_ref[...] = ps / jnp.sum(ps, axis=1, keepdims=True)


def _tc_call(q, P):
    B = q.shape[0]
    nb = B // BR_
    return pl.pallas_call(
        _tc_body,
        grid=(nb,),
        in_specs=[
            pl.BlockSpec((BR_, EMBED_), lambda i: (i, 0)),
            pl.BlockSpec((M_, EMBED_), lambda i: (0, 0)),
        ],
        out_specs=[
            pl.BlockSpec((BR_, NP_), lambda i: (i, 0)),
            pl.BlockSpec((BR_, M_), lambda i: (i, 0)),
        ],
        out_shape=[
            jax.ShapeDtypeStruct((B, NP_), jnp.float32),
            jax.ShapeDtypeStruct((B, M_), jnp.float32),
        ],
        compiler_params=pltpu.CompilerParams(
            dimension_semantics=("parallel",)),
    )(q, P)


_NC = 2
_NS = 16
_NW = _NC * _NS
_LANES = 16
_CH = 32768


def _make_sc_exp(N):
    per_w = N // _NW
    nch = per_w // _CH
    mesh = plsc.VectorSubcoreMesh(core_axis_name="c", subcore_axis_name="s")

    @functools.partial(
        pl.kernel, mesh=mesh,
        out_type=jax.ShapeDtypeStruct((N,), jnp.float32),
        scratch_types=[pltpu.VMEM((_CH,), jnp.float32)],
    )
    def sc_exp(d_hbm, e_hbm, buf):
        wid = lax.axis_index("s") * _NC + lax.axis_index("c")
        base = pl.multiple_of(wid * per_w, _CH)
        for c in range(nch):
            off = base + c * _CH
            pltpu.sync_copy(d_hbm.at[pl.ds(off, _CH)], buf)

            def body(j, carry):
                sl = pl.ds(pl.multiple_of(j * _LANES, _LANES), _LANES)
                buf[sl] = jnp.exp(-buf[sl])
                return carry

            lax.fori_loop(0, _CH // _LANES, body, 0)
            pltpu.sync_copy(buf, e_hbm.at[pl.ds(off, _CH)])

    return sc_exp


@functools.partial(jax.jit, static_argnames=())
def kernel(q, P):
    B = q.shape[0]
    pred, D = _tc_call(q, P)
    E = _make_sc_exp(B * M_)(D.reshape(-1))
    return (pred, D, E.reshape(B, NP_, NK_))


# speedup vs baseline: 1.9246x; 1.9246x over previous
import functools

import jax
import jax.numpy as jnp
from jax import lax
from jax.experimental import pallas as pl
from jax.experimental.pallas import tpu as pltpu
from jax.experimental.pallas import tpu_sc as plsc

NP_ = 32
NK_ = 32
M_ = NP_ * NK_
EMBED_ = 256
BR_ = 2048


def _tc_body(q_ref, P_ref, pred_ref, D_ref):
    q = q_ref[...]
    Pm = P_ref[...]
    qb = (q * -2.0).astype(jnp.bfloat16)
    Pb = Pm.astype(jnp.bfloat16)
    S = jax.lax.dot_general(
        qb, Pb, (((1,), (1,)), ((), ())),
        preferred_element_type=jnp.float32)
    q2 = jnp.sum(q * q, axis=1, keepdims=True)
    p2 = jnp.sum(Pm * Pm, axis=1)[None, :]
    m = jnp.maximum((q2 + p2) + S, 1e-12)
    D = m * jax.lax.rsqrt(m)
    E = jnp.exp(-D)
    D_ref[...] = D
    col = jax.lax.broadcasted_iota(jnp.int32, (M_, NP_), 0)
    grp = jax.lax.broadcasted_iota(jnp.int32, (M_, NP_), 1)
    G = jnp.where(col // NK_ == grp, 1.0, 0.0).astype(jnp.float32)
    ps = jax.lax.dot_general(
        E, G, (((1,), (0,)), ((), ())),
        preferred_element_type=jnp.float32)
    pred_ref[...] = ps / jnp.sum(ps, axis=1, keepdims=True)


def _tc_call(q, P):
    B = q.shape[0]
    nb = B // BR_
    return pl.pallas_call(
        _tc_body,
        grid=(nb,),
        in_specs=[
            pl.BlockSpec((BR_, EMBED_), lambda i: (i, 0)),
            pl.BlockSpec((M_, EMBED_), lambda i: (0, 0)),
        ],
        out_specs=[
            pl.BlockSpec((BR_, NP_), lambda i: (i, 0)),
            pl.BlockSpec((BR_, M_), lambda i: (i, 0)),
        ],
        out_shape=[
            jax.ShapeDtypeStruct((B, NP_), jnp.float32),
            jax.ShapeDtypeStruct((B, M_), jnp.float32),
        ],
        compiler_params=pltpu.CompilerParams(
            dimension_semantics=("parallel",)),
    )(q, P)


_NC = 2
_NS = 16
_NW = _NC * _NS
_LANES = 16
_CH = 32768


_NBUF = 3
_UNROLL = 8


def _make_sc_exp(N):
    per_w = N // _NW
    nch = per_w // _CH
    mesh = plsc.VectorSubcoreMesh(core_axis_name="c", subcore_axis_name="s")

    @functools.partial(
        pl.kernel, mesh=mesh,
        out_type=jax.ShapeDtypeStruct((N,), jnp.float32),
        scratch_types=(
            [pltpu.VMEM((_CH,), jnp.float32) for _ in range(_NBUF)]
            + [pltpu.SemaphoreType.DMA for _ in range(2 * _NBUF)]
        ),
    )
    def sc_exp(d_hbm, e_hbm, *scratch):
        bufs = scratch[:_NBUF]
        sem_in = scratch[_NBUF:2 * _NBUF]
        sem_out = scratch[2 * _NBUF:]
        wid = lax.axis_index("s") * _NC + lax.axis_index("c")
        base = pl.multiple_of(wid * per_w, _CH)

        def start_in(c):
            return pltpu.async_copy(
                d_hbm.at[pl.ds(base + c * _CH, _CH)], bufs[c % _NBUF],
                sem_in[c % _NBUF])

        def start_out(c):
            return pltpu.async_copy(
                bufs[c % _NBUF], e_hbm.at[pl.ds(base + c * _CH, _CH)],
                sem_out[c % _NBUF])

        def compute(buf):
            def body(j, carry):
                j0 = pl.multiple_of(j * (_LANES * _UNROLL), _LANES * _UNROLL)
                for k in range(_UNROLL):
                    sl = pl.ds(j0 + k * _LANES, _LANES)
                    buf[sl] = jnp.exp(-buf[sl])
                return carry

            lax.fori_loop(0, _CH // (_LANES * _UNROLL), body, 0)

        h_in = {}
        h_out = {}
        h_in[0] = start_in(0)
        for c in range(nch):
            if c + 1 < nch:
                if c + 1 >= _NBUF:
                    h_out[c + 1 - _NBUF].wait()
                h_in[c + 1] = start_in(c + 1)
            h_in[c].wait()
            compute(bufs[c % _NBUF])
            h_out[c] = start_out(c)
        for c in range(max(0, nch - _NBUF), nch):
            h_out[c].wait()

    return sc_exp


@functools.partial(jax.jit, static_argnames=())
def kernel(q, P):
    B = q.shape[0]
    pred, D = _tc_call(q, P)
    E = _make_sc_exp(B * M_)(D.reshape(-1))
    return (pred, D, E.reshape(B, NP_, NK_))


# speedup vs baseline: 4.3304x; 2.2501x over previous
import functools

import jax
import jax.numpy as jnp
from jax import lax
from jax.experimental import pallas as pl
from jax.experimental.pallas import tpu as pltpu
from jax.experimental.pallas import tpu_sc as plsc

NP_ = 32
NK_ = 32
M_ = NP_ * NK_
EMBED_ = 256
BR_ = 2048


def _tc_body(q_ref, P_ref, pred_ref, D_ref):
    q = q_ref[...]
    Pm = P_ref[...]
    qb = (q * -2.0).astype(jnp.bfloat16)
    Pb = Pm.astype(jnp.bfloat16)
    S = jax.lax.dot_general(
        qb, Pb, (((1,), (1,)), ((), ())),
        preferred_element_type=jnp.float32)
    q2 = jnp.sum(q * q, axis=1, keepdims=True)
    p2 = jnp.sum(Pm * Pm, axis=1)[None, :]
    m = jnp.maximum((q2 + p2) + S, 1e-12)
    D = m * jax.lax.rsqrt(m)
    E = jnp.exp(-D)
    D_ref[...] = D
    col = jax.lax.broadcasted_iota(jnp.int32, (M_, NP_), 0)
    grp = jax.lax.broadcasted_iota(jnp.int32, (M_, NP_), 1)
    G = jnp.where(col // NK_ == grp, 1.0, 0.0).astype(jnp.float32)
    ps = jax.lax.dot_general(
        E, G, (((1,), (0,)), ((), ())),
        preferred_element_type=jnp.float32)
    pred_ref[...] = ps / jnp.sum(ps, axis=1, keepdims=True)


def _tc_call(q, P):
    B = q.shape[0]
    nb = B // BR_
    return pl.pallas_call(
        _tc_body,
        grid=(nb,),
        in_specs=[
            pl.BlockSpec((BR_, EMBED_), lambda i: (i, 0)),
            pl.BlockSpec((M_, EMBED_), lambda i: (0, 0)),
        ],
        out_specs=[
            pl.BlockSpec((BR_, NP_), lambda i: (i, 0)),
            pl.BlockSpec((BR_, M_), lambda i: (i, 0)),
        ],
        out_shape=[
            jax.ShapeDtypeStruct((B, NP_), jnp.float32),
            jax.ShapeDtypeStruct((B, M_), jnp.float32),
        ],
        compiler_params=pltpu.CompilerParams(
            dimension_semantics=("parallel",)),
    )(q, P)


_NC = 2
_NS = 16
_NW = _NC * _NS
_LANES = 16
_CH = 32768


_NBUF = 3
_CHR = 32


def _make_sc_exp(B):
    rows_w = B // _NW
    nch = rows_w // _CHR
    mesh = plsc.VectorSubcoreMesh(core_axis_name="c", subcore_axis_name="s")

    @functools.partial(
        pl.kernel, mesh=mesh,
        out_type=jax.ShapeDtypeStruct((B, M_), jnp.float32),
        scratch_types=(
            [pltpu.VMEM((_CHR, M_), jnp.float32) for _ in range(_NBUF)]
            + [pltpu.SemaphoreType.DMA for _ in range(2 * _NBUF)]
        ),
        compiler_params=pltpu.CompilerParams(use_tc_tiling_on_sc=True),
    )
    def sc_exp(d_hbm, e_hbm, *scratch):
        bufs = scratch[:_NBUF]
        sem_in = scratch[_NBUF:2 * _NBUF]
        sem_out = scratch[2 * _NBUF:]
        wid = lax.axis_index("s") * _NC + lax.axis_index("c")
        base = pl.multiple_of(wid * rows_w, _CHR)

        def start_in(c):
            return pltpu.async_copy(
                d_hbm.at[pl.ds(base + c * _CHR, _CHR), :], bufs[c % _NBUF],
                sem_in[c % _NBUF])

        def start_out(c):
            return pltpu.async_copy(
                bufs[c % _NBUF], e_hbm.at[pl.ds(base + c * _CHR, _CHR), :],
                sem_out[c % _NBUF])

        def compute(buf):
            def body(i, carry):
                for k in range(M_ // _LANES):
                    sl = pl.ds(k * _LANES, _LANES)
                    buf[i, sl] = jnp.exp(-buf[i, sl])
                return carry

            lax.fori_loop(0, _CHR, body, 0)

        h_in = {}
        h_out = {}
        h_in[0] = start_in(0)
        for c in range(nch):
            if c + 1 < nch:
                if c + 1 >= _NBUF:
                    h_out[c + 1 - _NBUF].wait()
                h_in[c + 1] = start_in(c + 1)
            h_in[c].wait()
            compute(bufs[c % _NBUF])
            h_out[c] = start_out(c)
        for c in range(max(0, nch - _NBUF), nch):
            h_out[c].wait()

    return sc_exp


@functools.partial(jax.jit, static_argnames=())
def kernel(q, P):
    B = q.shape[0]
    pred, D = _tc_call(q, P)
    E = _make_sc_exp(B)(D)
    return (pred, D, E.reshape(B, NP_, NK_))
